# baseline (speedup 1.0000x reference)
import functools

import jax
import jax.numpy as jnp
from jax import lax
from jax.experimental import pallas as pl
from jax.experimental.pallas import tpu as pltpu
from jax.experimental.pallas import tpu_sc as plsc

_NUM_CORES = 2
_NUM_SUBCORES = 16
_NUM_WORKERS = _NUM_CORES * _NUM_SUBCORES

_CHUNK = 128


@functools.partial(jax.jit, static_argnames=("n_chunks", "embed_dim"))
def _sc_gather(ids3, table, *, n_chunks, embed_dim):
  n_rows = _NUM_WORKERS * n_chunks * _CHUNK
  rows_per_worker = n_chunks * _CHUNK

  mesh = plsc.VectorSubcoreMesh(
      core_axis_name="c", subcore_axis_name="s",
      num_cores=_NUM_CORES, num_subcores=_NUM_SUBCORES)

  @functools.partial(
      pl.kernel,
      out_type=jax.ShapeDtypeStruct((n_rows, embed_dim), jnp.float32),
      mesh=mesh,
      scratch_types=[
          pltpu.VMEM((n_chunks, _CHUNK), jnp.int32),
          pltpu.VMEM((_CHUNK, embed_dim), jnp.float32),
          pltpu.SemaphoreType.DMA,
      ],
      compiler_params=pltpu.CompilerParams(use_tc_tiling_on_sc=False),
  )
  def gather_kernel(ids_hbm, table_hbm, out_hbm, idx_v, rows_v, sem):
    wid = lax.axis_index("s") * _NUM_CORES + lax.axis_index("c")
    base = wid * rows_per_worker
    pltpu.sync_copy(ids_hbm.at[wid], idx_v)

    def body(j, carry):
      pltpu.async_copy(table_hbm.at[idx_v.at[j]], rows_v, sem).wait()
      pltpu.sync_copy(rows_v, out_hbm.at[pl.ds(base + j * _CHUNK, _CHUNK)])
      return carry

    lax.fori_loop(0, n_chunks, body, 0)

  return gather_kernel(ids3, table)


def kernel(ids, table, mask):
  b, t = ids.shape
  vocab, embed_dim = table.shape
  n = b * t
  assert n % (_NUM_WORKERS * _CHUNK) == 0
  n_chunks = n // (_NUM_WORKERS * _CHUNK)
  ids3 = ids.reshape(_NUM_WORKERS, n_chunks, _CHUNK)
  rows = _sc_gather(ids3, table, n_chunks=n_chunks, embed_dim=embed_dim)
  return (rows.reshape(b, t, embed_dim), mask)


# speedup vs baseline: 1.0431x; 1.0431x over previous
import functools

import jax
import jax.numpy as jnp
from jax import lax
from jax.experimental import pallas as pl
from jax.experimental.pallas import tpu as pltpu
from jax.experimental.pallas import tpu_sc as plsc

_NUM_CORES = 2
_NUM_SUBCORES = 16
_NUM_WORKERS = _NUM_CORES * _NUM_SUBCORES

_CHUNK = 128
_K = 5


@functools.partial(jax.jit, static_argnames=("n_chunks", "embed_dim"))
def _sc_gather(ids3, table, *, n_chunks, embed_dim):
  n_rows = _NUM_WORKERS * n_chunks * _CHUNK
  rows_per_worker = n_chunks * _CHUNK
  n_groups = n_chunks // _K
  assert n_chunks % _K == 0 and n_groups % 2 == 0
  n_pairs = n_groups // 2

  mesh = plsc.VectorSubcoreMesh(
      core_axis_name="c", subcore_axis_name="s",
      num_cores=_NUM_CORES, num_subcores=_NUM_SUBCORES)

  @functools.partial(
      pl.kernel,
      out_type=jax.ShapeDtypeStruct((n_rows, embed_dim), jnp.float32),
      mesh=mesh,
      scratch_types=[
          pltpu.VMEM((n_chunks, _CHUNK), jnp.int32),
          pltpu.VMEM((2, _K, _CHUNK, embed_dim), jnp.float32),
          pltpu.SemaphoreType.DMA,
          pltpu.SemaphoreType.DMA,
          pltpu.SemaphoreType.DMA,
          pltpu.SemaphoreType.DMA,
      ],
      compiler_params=pltpu.CompilerParams(use_tc_tiling_on_sc=False),
  )
  def gather_kernel(ids_hbm, table_hbm, out_hbm, idx_v, rows_v,
                    gsem0, gsem1, wsem0, wsem1):
    wid = lax.axis_index("s") * _NUM_CORES + lax.axis_index("c")
    base = wid * rows_per_worker
    gsem = (gsem0, gsem1)
    wsem = (wsem0, wsem1)

    pltpu.sync_copy(ids_hbm.at[wid], idx_v)

    def g_copy(h, b, g):
      j = g * _K + b
      return pltpu.make_async_copy(
          table_hbm.at[idx_v.at[j]], rows_v.at[h, b], gsem[h])

    def w_copy(h, b, g):
      j = g * _K + b
      return pltpu.make_async_copy(
          rows_v.at[h, b], out_hbm.at[pl.ds(base + j * _CHUNK, _CHUNK)],
          wsem[h])

    def fire_g(h, g):
      for b in range(_K):
        g_copy(h, b, g).start()

    def drain_g(h, g):
      for b in range(_K):
        g_copy(h, b, g).wait()

    def fire_w(h, g):
      for b in range(_K):
        w_copy(h, b, g).start()

    def drain_w(h, g):
      for b in range(_K):
        w_copy(h, b, g).wait()

    fire_g(0, 0)

    def body(p, carry):
      g0 = 2 * p
      drain_g(0, g0)
      pl.when(p > 0)(lambda: drain_w(1, g0 - 1))
      fire_g(1, g0 + 1)
      fire_w(0, g0)
      drain_g(1, g0 + 1)
      drain_w(0, g0)
      pl.when(p < n_pairs - 1)(lambda: fire_g(0, g0 + 2))
      fire_w(1, g0 + 1)
      return carry

    lax.fori_loop(0, n_pairs, body, 0)
    drain_w(1, n_groups - 1)

  return gather_kernel(ids3, table)


def kernel(ids, table, mask):
  b, t = ids.shape
  vocab, embed_dim = table.shape
  n = b * t
  assert n % (_NUM_WORKERS * _CHUNK) == 0
  n_chunks = n // (_NUM_WORKERS * _CHUNK)
  ids3 = ids.reshape(_NUM_WORKERS, n_chunks, _CHUNK)
  rows = _sc_gather(ids3, table, n_chunks=n_chunks, embed_dim=embed_dim)
  return (rows.reshape(b, t, embed_dim), mask)


# speedup vs baseline: 1.0953x; 1.0501x over previous
import functools

import jax
import jax.numpy as jnp
from jax import lax
from jax.experimental import pallas as pl
from jax.experimental.pallas import tpu as pltpu
from jax.experimental.pallas import tpu_sc as plsc

_NUM_CORES = 2
_NUM_SUBCORES = 16
_NUM_WORKERS = _NUM_CORES * _NUM_SUBCORES

_CHUNK = 64
_K = 5
_PAD_DIM = 128


@functools.partial(jax.jit, static_argnames=("n_chunks", "embed_dim"))
def _sc_gather(ids3, table, *, n_chunks, embed_dim):
  n_rows = _NUM_WORKERS * n_chunks * _CHUNK
  rows_per_worker = n_chunks * _CHUNK
  n_groups = n_chunks // _K
  assert n_chunks % _K == 0 and n_groups % 2 == 0
  n_pairs = n_groups // 2

  mesh = plsc.VectorSubcoreMesh(
      core_axis_name="c", subcore_axis_name="s",
      num_cores=_NUM_CORES, num_subcores=_NUM_SUBCORES)

  @functools.partial(
      pl.kernel,
      out_type=jax.ShapeDtypeStruct((n_rows, embed_dim), jnp.float32),
      mesh=mesh,
      scratch_types=[
          pltpu.VMEM((n_chunks, _CHUNK), jnp.int32),
          pltpu.VMEM((2, _K, _CHUNK, _PAD_DIM), jnp.float32),
          pltpu.SemaphoreType.DMA,
          pltpu.SemaphoreType.DMA,
          pltpu.SemaphoreType.DMA,
          pltpu.SemaphoreType.DMA,
      ],
      compiler_params=pltpu.CompilerParams(use_tc_tiling_on_sc=False),
  )
  def gather_kernel(ids_hbm, table_hbm, out_hbm, idx_v, rows_v,
                    gsem0, gsem1, wsem0, wsem1):
    wid = lax.axis_index("s") * _NUM_CORES + lax.axis_index("c")
    base = wid * rows_per_worker
    gsem = (gsem0, gsem1)
    wsem = (wsem0, wsem1)

    pltpu.sync_copy(ids_hbm.at[wid], idx_v)

    def g_copy(h, b, g):
      j = g * _K + b
      return pltpu.make_async_copy(
          table_hbm.at[idx_v.at[j]], rows_v.at[h, b], gsem[h])

    def w_copy(h, b, g):
      j = g * _K + b
      return pltpu.make_async_copy(
          rows_v.at[h, b, :, pl.ds(0, embed_dim)],
          out_hbm.at[pl.ds(base + j * _CHUNK, _CHUNK)],
          wsem[h])

    def fire_g(h, g):
      for b in range(_K):
        g_copy(h, b, g).start()

    def drain_g(h, g):
      for b in range(_K):
        g_copy(h, b, g).wait()

    def fire_w(h, g):
      for b in range(_K):
        w_copy(h, b, g).start()

    def drain_w(h, g):
      for b in range(_K):
        w_copy(h, b, g).wait()

    fire_g(0, 0)

    def body(p, carry):
      g0 = 2 * p
      drain_g(0, g0)
      pl.when(p > 0)(lambda: drain_w(1, g0 - 1))
      fire_g(1, g0 + 1)
      fire_w(0, g0)
      drain_g(1, g0 + 1)
      drain_w(0, g0)
      pl.when(p < n_pairs - 1)(lambda: fire_g(0, g0 + 2))
      fire_w(1, g0 + 1)
      return carry

    lax.fori_loop(0, n_pairs, body, 0)
    drain_w(1, n_groups - 1)

  return gather_kernel(ids3, table)


def kernel(ids, table, mask):
  b, t = ids.shape
  vocab, embed_dim = table.shape
  n = b * t
  assert n % (_NUM_WORKERS * _CHUNK) == 0
  n_chunks = n // (_NUM_WORKERS * _CHUNK)
  ids3 = ids.reshape(_NUM_WORKERS, n_chunks, _CHUNK)
  table_pad = jnp.pad(table, ((0, 0), (0, _PAD_DIM - embed_dim)))
  rows = _sc_gather(ids3, table_pad, n_chunks=n_chunks, embed_dim=embed_dim)
  return (rows.reshape(b, t, embed_dim), mask)
